# Initial kernel scaffold; baseline (speedup 1.0000x reference)
import jax
import jax.numpy as jnp
from jax.experimental import pallas as pl


def kernel(x1, x2, mask, Wq, Wk, Wv, Wo, rot, W1, b1, W2, b2, g1, be1, g2, be2):
    raise NotImplementedError("write your pallas kernel here")


# speedup vs baseline: 7.0796x; 7.0796x over previous
import functools

import jax
import jax.numpy as jnp
from jax import lax
from jax.experimental import pallas as pl
from jax.experimental.pallas import tpu as pltpu, tpu_sc as plsc

N_LAYERS = 2
D = 1024
H = 16
DH = 64
FF = 4096
ROUNDS = 2
CHUNK = 64
NB = 128
B = 2
S = 4096
PW = 256
OW = 128
RBH = ROUNDS * B * H
NCH = S // CHUNK
CSUB = 32


_TSA = 256


def _qkv_body(x_ref, w_ref, o_ref):
    x = x_ref[0]
    w = w_ref[0]
    y = jnp.dot(x, w, preferred_element_type=jnp.float32)
    st = pl.program_id(1)
    pos = (lax.broadcasted_iota(jnp.int32, (_TSA, 16), 0)
           + st * _TSA).astype(jnp.float32)
    pad = jnp.zeros((_TSA, PW - 3 * DH - 16), jnp.float32)
    o_ref[0, 0] = jnp.concatenate([y, pos, pad], axis=-1)


def _qkv_call(x2, wh):
    return pl.pallas_call(
        _qkv_body,
        grid=(B, S // _TSA, H),
        in_specs=[
            pl.BlockSpec((1, _TSA, D), lambda b, s, h: (b, s, 0)),
            pl.BlockSpec((1, D, 3 * DH), lambda b, s, h: (h, 0, 0)),
        ],
        out_specs=pl.BlockSpec((1, 1, _TSA, PW), lambda b, s, h: (b, h, s, 0)),
        out_shape=jax.ShapeDtypeStruct((B, H, S, PW), jnp.float32),
    )(x2, wh)


def _route_body(q_ref, rot_ref, o_ref):
    q = q_ref[0, 0][:, :DH]
    rt = rot_ref[0, 0]
    rq = jnp.dot(q, rt, preferred_element_type=jnp.float32)
    sc = jnp.concatenate([rq, -rq], axis=-1)
    m = jnp.max(sc, axis=-1, keepdims=True)
    io = lax.broadcasted_iota(jnp.int32, (S, NB), 1)
    bk = jnp.min(jnp.where(sc == m, io, NB), axis=-1, keepdims=True)
    z = (io == bk).astype(jnp.float32)
    z3 = z.reshape(CSUB, S // CSUB, NB)

    rr = lax.broadcasted_iota(jnp.int32, (S // CSUB, S // CSUB), 0)
    cc = lax.broadcasted_iota(jnp.int32, (S // CSUB, S // CSUB), 1)
    tri = (rr >= cc).astype(jnp.float32)
    cum = jnp.stack(
        [jnp.dot(tri, z3[c], preferred_element_type=jnp.float32)
         for c in range(CSUB)], axis=0)
    ctot = jnp.sum(z3, axis=1)
    r2 = lax.broadcasted_iota(jnp.int32, (CSUB, CSUB), 0)
    c2 = lax.broadcasted_iota(jnp.int32, (CSUB, CSUB), 1)
    tri_s = (r2 > c2).astype(jnp.float32)
    excl = jnp.dot(tri_s, ctot, preferred_element_type=jnp.float32)
    fullcum = cum + excl[:, None, :]
    rank = jnp.sum(z3 * fullcum, axis=-1)
    counts = jnp.sum(ctot, axis=0, keepdims=True)
    r3 = lax.broadcasted_iota(jnp.int32, (NB, NB), 0)
    c3 = lax.broadcasted_iota(jnp.int32, (NB, NB), 1)
    tri_u = (r3 < c3).astype(jnp.float32)
    off = jnp.dot(counts, tri_u, preferred_element_type=jnp.float32)
    offat = jnp.sum(z3 * off[None], axis=-1)
    ui = offat + rank - 1.0
    job = (pl.program_id(0) * B + pl.program_id(1)) * H + pl.program_id(2)
    o_ref[0, 0, 0] = ui.astype(jnp.int32) + job * S


def _route_call(qkvp, rot_l):
    return pl.pallas_call(
        _route_body,
        grid=(ROUNDS, B, H),
        in_specs=[
            pl.BlockSpec((1, 1, S, PW), lambda r, b, h: (b, h, 0, 0)),
            pl.BlockSpec((1, 1, DH, NB // 2), lambda r, b, h: (r, h, 0, 0)),
        ],
        out_specs=pl.BlockSpec((1, 1, 1, CSUB, S // CSUB),
                               lambda r, b, h: (r, b, h, 0, 0)),
        out_shape=jax.ShapeDtypeStruct((ROUNDS, B, H, CSUB, S // CSUB),
                                       jnp.int32),
    )(qkvp, rot_l)


def _sc_move(table, idxg, gather):
    width = table.shape[1]
    info = plsc.get_sparse_core_info()
    nc, ns = info.num_cores, info.num_subcores
    nw = nc * ns
    rpw = S // nw
    mesh = plsc.VectorSubcoreMesh(core_axis_name="c", subcore_axis_name="s")

    @functools.partial(
        pl.kernel,
        out_type=jax.ShapeDtypeStruct((RBH * S, width), jnp.float32),
        mesh=mesh,
        scratch_types=[
            pltpu.VMEM((rpw,), jnp.int32),
            pltpu.VMEM((rpw, width), jnp.float32),
            pltpu.SemaphoreType.DMA,
        ],
    )
    def mover(tab_hbm, idx_hbm, out_hbm, idx_v, rows_v, sem):
        wid = lax.axis_index("s") * nc + lax.axis_index("c")

        def step(j, carry):
            base = j * S + wid * rpw
            pltpu.sync_copy(idx_hbm.at[pl.ds(base, rpw)], idx_v)
            if gather:
                pltpu.async_copy(tab_hbm.at[idx_v], rows_v, sem).wait()
                pltpu.sync_copy(rows_v, out_hbm.at[pl.ds(base, rpw)])
            else:
                src = lax.rem(j, B * H) * S + wid * rpw
                pltpu.sync_copy(tab_hbm.at[pl.ds(src, rpw)], rows_v)
                pltpu.async_copy(rows_v, out_hbm.at[idx_v], sem).wait()
            return carry

        lax.fori_loop(0, RBH, step, 0)

    return mover(table, idxg)


def _attn_body(s_ref, o_ref):
    buf = s_ref[0]
    q = buf[:, :DH] * (1.0 / 8.0)
    k = buf[:, DH:2 * DH]
    v = buf[:, 2 * DH:3 * DH]
    p = buf[:, 3 * DH:3 * DH + 1]
    ones = jnp.ones((CHUNK, 1), jnp.float32)
    dn = (((1,), (1,)), ((), ()))
    for c in range(NCH):
        pc = (c - 1) % NCH
        sl = slice(c * CHUNK, (c + 1) * CHUNK)
        slp = slice(pc * CHUNK, (pc + 1) * CHUNK)
        qc = q[sl, :]
        kc = jnp.concatenate([k[slp, :], k[sl, :]], axis=0)
        vc = jnp.concatenate([v[slp, :], v[sl, :]], axis=0)
        pq = p[sl, :]
        pk = jnp.concatenate([p[slp, :], p[sl, :]], axis=0)
        pkmat = lax.dot_general(ones, pk, dn,
                                preferred_element_type=jnp.float32)
        sc = lax.dot_general(qc, kc, dn,
                             preferred_element_type=jnp.float32)
        sc = jnp.where(pkmat <= pq, sc, -1e9)
        m = jnp.max(sc, axis=-1, keepdims=True)
        e = jnp.exp(sc - m)
        den = jnp.sum(e, axis=-1, keepdims=True)
        o = jnp.dot(e, vc, preferred_element_type=jnp.float32) / den
        lse = jnp.log(den) + m
        o_ref[0, sl, :] = jnp.concatenate(
            [o, jnp.broadcast_to(lse, (CHUNK, 16)),
             jnp.zeros((CHUNK, OW - DH - 16), jnp.float32)], axis=-1)


def _attn_call(sorted3):
    return pl.pallas_call(
        _attn_body,
        grid=(RBH,),
        in_specs=[pl.BlockSpec((1, S, PW), lambda j: (j, 0, 0))],
        out_specs=pl.BlockSpec((1, S, OW), lambda j: (j, 0, 0)),
        out_shape=jax.ShapeDtypeStruct((RBH, S, OW), jnp.float32),
    )(sorted3)


_TSF = 256


def _combine_body(un_ref, x1_ref, wo_ref, g_ref, be_ref, o_ref):
    heads = []
    for h in range(H):
        o0 = un_ref[0, 0, h, :, :DH]
        l0 = un_ref[0, 0, h, :, DH:DH + 1]
        o1 = un_ref[1, 0, h, :, :DH]
        l1 = un_ref[1, 0, h, :, DH:DH + 1]
        w0 = 1.0 / (1.0 + jnp.exp(l1 - l0))
        heads.append(o0 * w0 + o1 * (1.0 - w0))
    a = jnp.concatenate(heads, axis=-1)
    a = jnp.dot(a, wo_ref[...], preferred_element_type=jnp.float32)
    m = jnp.mean(a, axis=-1, keepdims=True)
    vv = jnp.mean((a - m) ** 2, axis=-1, keepdims=True)
    ln = (a - m) / jnp.sqrt(vv + 1e-5) * g_ref[...] + be_ref[...]
    o_ref[0] = x1_ref[0] + ln


def _combine_call(un, x1, wo, g1, be1):
    return pl.pallas_call(
        _combine_body,
        grid=(B, S // _TSF),
        in_specs=[
            pl.BlockSpec((ROUNDS, 1, H, _TSF, OW), lambda b, s: (0, b, 0, s, 0)),
            pl.BlockSpec((1, _TSF, D), lambda b, s: (b, s, 0)),
            pl.BlockSpec((D, D), lambda b, s: (0, 0)),
            pl.BlockSpec((1, D), lambda b, s: (0, 0)),
            pl.BlockSpec((1, D), lambda b, s: (0, 0)),
        ],
        out_specs=pl.BlockSpec((1, _TSF, D), lambda b, s: (b, s, 0)),
        out_shape=jax.ShapeDtypeStruct((B, S, D), jnp.float32),
    )(un, x1, wo, g1.reshape(1, D), be1.reshape(1, D))


_TSG = 256


def _ffn_body(y1_ref, x2_ref, w1_ref, b1_ref, w2_ref, b2_ref, g_ref, be_ref,
              o_ref):
    y = y1_ref[0]
    hmid = jnp.maximum(
        jnp.dot(y, w1_ref[...], preferred_element_type=jnp.float32)
        + b1_ref[...], 0.0)
    f = jnp.dot(hmid, w2_ref[...], preferred_element_type=jnp.float32) \
        + b2_ref[...]
    m = jnp.mean(f, axis=-1, keepdims=True)
    vv = jnp.mean((f - m) ** 2, axis=-1, keepdims=True)
    ln = (f - m) / jnp.sqrt(vv + 1e-5) * g_ref[...] + be_ref[...]
    o_ref[0] = x2_ref[0] + ln


def _ffn_call(y1, x2, w1, b1, w2, b2, g2, be2):
    return pl.pallas_call(
        _ffn_body,
        grid=(B, S // _TSG),
        in_specs=[
            pl.BlockSpec((1, _TSG, D), lambda b, s: (b, s, 0)),
            pl.BlockSpec((1, _TSG, D), lambda b, s: (b, s, 0)),
            pl.BlockSpec((D, FF), lambda b, s: (0, 0)),
            pl.BlockSpec((1, FF), lambda b, s: (0, 0)),
            pl.BlockSpec((FF, D), lambda b, s: (0, 0)),
            pl.BlockSpec((1, D), lambda b, s: (0, 0)),
            pl.BlockSpec((1, D), lambda b, s: (0, 0)),
            pl.BlockSpec((1, D), lambda b, s: (0, 0)),
        ],
        out_specs=pl.BlockSpec((1, _TSG, D), lambda b, s: (b, s, 0)),
        out_shape=jax.ShapeDtypeStruct((B, S, D), jnp.float32),
    )(y1, x2, w1, b1.reshape(1, FF), w2, b2.reshape(1, D),
      g2.reshape(1, D), be2.reshape(1, D))


def _layer(x1, x2, wq, wk, wv, wo, rot_l, w1, b1, w2, b2, g1, be1, g2, be2):
    wh = jnp.concatenate(
        [wq.reshape(D, H, DH).transpose(1, 0, 2),
         wk.reshape(D, H, DH).transpose(1, 0, 2),
         wv.reshape(D, H, DH).transpose(1, 0, 2)], axis=-1)
    qkvp = _qkv_call(x2, wh)
    uig = _route_call(qkvp, rot_l)
    uig_flat = uig.reshape(RBH * S)
    sorted_buf = _sc_move(qkvp.reshape(B * H * S, PW), uig_flat, False)
    att = _attn_call(sorted_buf.reshape(RBH, S, PW))
    un = _sc_move(att.reshape(RBH * S, OW), uig_flat, True)
    y1 = _combine_call(un.reshape(ROUNDS, B, H, S, OW), x1, wo, g1, be1)
    y2 = _ffn_call(y1, x2, w1, b1, w2, b2, g2, be2)
    return y1, y2


def kernel(x1, x2, mask, Wq, Wk, Wv, Wo, rot, W1, b1, W2, b2, g1, be1, g2,
           be2):
    del mask
    for i in range(N_LAYERS):
        x1, x2 = _layer(x1, x2, Wq[i], Wk[i], Wv[i], Wo[i], rot[i], W1[i],
                        b1[i], W2[i], b2[i], g1[i], be1[i], g2[i], be2[i])
    return x2
